# baseline (speedup 1.0000x reference)
import functools

import jax
import jax.numpy as jnp
from jax import lax
from jax.experimental import pallas as pl
from jax.experimental.pallas import tpu as pltpu
from jax.experimental.pallas import tpu_sc as plsc

D_MODEL = 64
NUM_CORES = 2
NUM_SUBCORES = 16
NUM_WORKERS = NUM_CORES * NUM_SUBCORES
CHUNK = 128
NBUF = 2
UNROLL = 16
K_CALLS = 4
BI = 2048


def _emb_call(B):
    n_chunks = B // (NUM_WORKERS * CHUNK)
    mesh = plsc.VectorSubcoreMesh(core_axis_name="c", subcore_axis_name="s")

    @functools.partial(
        pl.kernel,
        mesh=mesh,
        compiler_params=pltpu.CompilerParams(use_tc_tiling_on_sc=True),
        out_type=jax.ShapeDtypeStruct((B, D_MODEL), jnp.float32),
        scratch_types=[
            pltpu.VMEM((n_chunks, CHUNK), jnp.int32),
            pltpu.VMEM((n_chunks, CHUNK), jnp.int32),
            pltpu.VMEM((NBUF, CHUNK, 2 * D_MODEL), jnp.float32),
            pltpu.VMEM((NBUF, CHUNK, D_MODEL), jnp.float32),
            pltpu.SemaphoreType.DMA((NBUF,)),
            pltpu.SemaphoreType.DMA((NBUF,)),
        ],
    )
    def emb(table_hbm, sidx_hbm, hoff_hbm, out_hbm,
            sidx_v, hoff_v, g_v, o_v, gsem, wsem):
        wid = lax.axis_index("s") * NUM_CORES + lax.axis_index("c")
        base = wid * n_chunks
        pltpu.sync_copy(sidx_hbm.at[wid], sidx_v)
        pltpu.sync_copy(hoff_hbm.at[wid], hoff_v)

        def gather(g, b):
            return pltpu.make_async_copy(
                table_hbm.at[sidx_v.at[g]], g_v.at[b], gsem.at[b])

        def out_write(g, b):
            return pltpu.make_async_copy(
                o_v.at[b], out_hbm.at[pl.ds((base + g) * CHUNK, CHUNK)],
                wsem.at[b])

        def compact(g, b):
            def blk_body(k, carry):
                kk = k * UNROLL
                hv = hoff_v[g, pl.ds(kk, UNROLL)]
                for r in range(UNROLL):
                    row = kk + r
                    h = hv[r]
                    for q in range(D_MODEL // 16):
                        o_v[b, row, pl.ds(q * 16, 16)] = (
                            g_v[b, row, pl.ds(h + q * 16, 16)])
                return carry

            lax.fori_loop(0, CHUNK // UNROLL, blk_body, 0)

        def step(g, b, wait_prev):
            gather(g, b).wait()
            if wait_prev:
                out_write(g - NBUF, b).wait()
            compact(g, b)
            out_write(g, b).start()

        def body(i, carry):
            for b in range(NBUF):
                g = i * NBUF + b
                step(g, b, True)
                gather(g + NBUF, b).start()
            return carry

        for b in range(NBUF):
            gather(b, b).start()
        for b in range(NBUF):
            step(b, b, False)
            gather(b + NBUF, b).start()
        n_outer = n_chunks // NBUF
        lax.fori_loop(1, n_outer - 1, body, 0)
        for b in range(NBUF):
            step((n_outer - 1) * NBUF + b, b, True)
        for b in range(NBUF):
            out_write((n_outer - 1) * NBUF + b, b).wait()

    return emb


def _transpose_call(j_per_call, n_tok, j_off, first):
    grid = (j_per_call, n_tok // BI)

    def tr(*refs):
        slab_ref, out_ref = refs[-2], refs[-1]
        out_ref[0] = slab_ref[0].T

    slab_spec = pl.BlockSpec((1, BI, D_MODEL), lambda j, g: (j, g, 0))
    out_spec = pl.BlockSpec((1, D_MODEL, BI), lambda j, g: (j_off + j, 0, g))
    out_shape = jax.ShapeDtypeStruct((200, D_MODEL, n_tok), jnp.float32)
    if first:
        return pl.pallas_call(
            tr, grid=grid, in_specs=[slab_spec], out_specs=out_spec,
            out_shape=out_shape)
    return pl.pallas_call(
        tr, grid=grid,
        in_specs=[pl.BlockSpec(memory_space=pl.ANY), slab_spec],
        out_specs=out_spec, out_shape=out_shape,
        input_output_aliases={0: 0})


def kernel(x, W):
    n_rows, seq = x.shape
    B = n_rows * seq
    b_call = B // K_CALLS
    j_per_call = seq // K_CALLS
    flat = x.T.astype(jnp.int32).reshape(-1)
    sidx = (flat >> 1).reshape(K_CALLS, NUM_WORKERS, -1, CHUNK)
    hoff = ((flat & 1) << 6).reshape(K_CALLS, NUM_WORKERS, -1, CHUNK)
    table = W.reshape(-1, 2 * D_MODEL)

    emb = _emb_call(b_call)
    slabs = [emb(table, sidx[c], hoff[c]) for c in range(K_CALLS)]

    out_t = _transpose_call(j_per_call, n_rows, 0, True)(
        slabs[0].reshape(j_per_call, n_rows, D_MODEL))
    for c in range(1, K_CALLS):
        out_t = _transpose_call(j_per_call, n_rows, c * j_per_call, False)(
            out_t, slabs[c].reshape(j_per_call, n_rows, D_MODEL))
    return out_t.transpose(2, 0, 1)


# speedup vs baseline: 1.0376x; 1.0376x over previous
import functools

import jax
import jax.numpy as jnp
from jax import lax
from jax.experimental import pallas as pl
from jax.experimental.pallas import tpu as pltpu
from jax.experimental.pallas import tpu_sc as plsc

D_MODEL = 64
NUM_CORES = 2
NUM_SUBCORES = 16
NUM_WORKERS = NUM_CORES * NUM_SUBCORES
CHUNK = 128
NBUF = 2
UNROLL = 16


def _emb_call(B):
    n_chunks = B // (NUM_WORKERS * CHUNK)
    mesh = plsc.VectorSubcoreMesh(core_axis_name="c", subcore_axis_name="s")

    @functools.partial(
        pl.kernel,
        mesh=mesh,
        compiler_params=pltpu.CompilerParams(use_tc_tiling_on_sc=True),
        out_type=jax.ShapeDtypeStruct((B, D_MODEL), jnp.float32),
        scratch_types=[
            pltpu.VMEM((n_chunks, CHUNK), jnp.int32),
            pltpu.VMEM((n_chunks, CHUNK), jnp.int32),
            pltpu.VMEM((NBUF, CHUNK, 2 * D_MODEL), jnp.float32),
            pltpu.VMEM((NBUF, CHUNK, D_MODEL), jnp.float32),
            pltpu.SemaphoreType.DMA((NBUF,)),
            pltpu.SemaphoreType.DMA((NBUF,)),
        ],
    )
    def emb(table_hbm, sidx_hbm, hoff_hbm, out_hbm,
            sidx_v, hoff_v, g_v, o_v, gsem, wsem):
        wid = lax.axis_index("s") * NUM_CORES + lax.axis_index("c")
        base = wid * n_chunks
        pltpu.sync_copy(sidx_hbm.at[wid], sidx_v)
        pltpu.sync_copy(hoff_hbm.at[wid], hoff_v)

        def gather(g, b):
            return pltpu.make_async_copy(
                table_hbm.at[sidx_v.at[g]], g_v.at[b], gsem.at[b])

        def out_write(g, b):
            return pltpu.make_async_copy(
                o_v.at[b], out_hbm.at[pl.ds((base + g) * CHUNK, CHUNK)],
                wsem.at[b])

        def compact(g, b):
            def blk_body(k, carry):
                kk = k * UNROLL
                hv = hoff_v[g, pl.ds(kk, UNROLL)]
                for r in range(UNROLL):
                    row = kk + r
                    h = hv[r]
                    for q in range(D_MODEL // 16):
                        o_v[b, row, pl.ds(q * 16, 16)] = (
                            g_v[b, row, pl.ds(h + q * 16, 16)])
                return carry

            lax.fori_loop(0, CHUNK // UNROLL, blk_body, 0)

        def step(g, b, wait_prev):
            gather(g, b).wait()
            if wait_prev:
                out_write(g - NBUF, b).wait()
            compact(g, b)
            out_write(g, b).start()

        for b in range(NBUF):
            gather(b, b).start()

        def body(i, carry):
            for b in range(NBUF):
                g = i * NBUF + b
                step(g, b, True)
                gather(g + NBUF, b).start()
            return carry

        for b in range(NBUF):
            step(b, b, False)
            gather(b + NBUF, b).start()
        n_outer = n_chunks // NBUF
        lax.fori_loop(1, n_outer - 1, body, 0)
        for b in range(NBUF):
            step((n_outer - 1) * NBUF + b, b, True)
        for b in range(NBUF):
            out_write((n_outer - 1) * NBUF + b, b).wait()

    return emb


def kernel(x, W):
    n_rows, seq = x.shape
    B = n_rows * seq
    xi = x.astype(jnp.int32).reshape(NUM_WORKERS, -1, CHUNK)
    sidx = xi >> 1
    hoff = (xi & 1) << 6
    table = W.reshape(-1, 2 * D_MODEL)
    out = _emb_call(B)(table, sidx, hoff)
    return out.reshape(n_rows, seq, D_MODEL)
